# baseline (device time: 114861 ns/iter reference)
import jax
import jax.numpy as jnp
from jax import lax
from jax.experimental import pallas as pl
from jax.experimental.pallas import tpu as pltpu

N_DEV = 8


def kernel(A, B):
    M, K = A.shape
    K2, N = B.shape
    assert K == K2
    CHUNK = M // N_DEV

    def body(a_ref, b_ref, out_ref, a_bf, b_bf, send_buf, recv_buf,
             send_sem, recv_sems):
        my = lax.axis_index("i")
        left = lax.rem(my + N_DEV - 1, N_DEV)
        right = lax.rem(my + 1, N_DEV)

        barrier_sem = pltpu.get_barrier_semaphore()
        for nbr in (left, right):
            pl.semaphore_signal(
                barrier_sem, inc=1,
                device_id=(nbr,), device_id_type=pl.DeviceIdType.MESH,
            )
        pl.semaphore_wait(barrier_sem, 2)

        a_bf[...] = a_ref[...].astype(jnp.bfloat16)
        b_bf[...] = b_ref[...].astype(jnp.bfloat16)

        def pchunk(c):
            a_c = a_bf[pl.ds(c * CHUNK, CHUNK), :]
            return lax.dot_general(
                a_c, b_bf[...], (((1,), (0,)), ((), ())),
                preferred_element_type=jnp.float32,
            )

        acc = pchunk(left)
        for s in range(N_DEV - 1):
            send_buf[...] = acc.astype(jnp.bfloat16)
            rdma = pltpu.make_async_remote_copy(
                src_ref=send_buf,
                dst_ref=recv_buf.at[s],
                send_sem=send_sem,
                recv_sem=recv_sems.at[s],
                device_id=(right,),
                device_id_type=pl.DeviceIdType.MESH,
            )
            rdma.start()
            rdma.wait()
            c = lax.rem(my + (2 * N_DEV - 2 - s), N_DEV)
            acc = recv_buf[s].astype(jnp.float32) + pchunk(c)
        out_ref[...] = acc

    return pl.pallas_call(
        body,
        out_shape=jax.ShapeDtypeStruct((CHUNK, N), jnp.float32),
        in_specs=[
            pl.BlockSpec(memory_space=pltpu.VMEM),
            pl.BlockSpec(memory_space=pltpu.VMEM),
        ],
        out_specs=pl.BlockSpec(memory_space=pltpu.VMEM),
        scratch_shapes=[
            pltpu.VMEM((M, K), jnp.bfloat16),
            pltpu.VMEM((K, N), jnp.bfloat16),
            pltpu.VMEM((CHUNK, N), jnp.bfloat16),
            pltpu.VMEM((N_DEV - 1, CHUNK, N), jnp.bfloat16),
            pltpu.SemaphoreType.DMA,
            pltpu.SemaphoreType.DMA((N_DEV - 1,)),
        ],
        compiler_params=pltpu.CompilerParams(collective_id=0),
    )(A, B)


# device time: 71091 ns/iter; 1.6157x vs baseline; 1.6157x over previous
import jax
import jax.numpy as jnp
from jax import lax
from jax.experimental import pallas as pl
from jax.experimental.pallas import tpu as pltpu

N_DEV = 8


def kernel(A, B):
    M, K = A.shape
    K2, N = B.shape
    assert K == K2
    CHUNK = M // N_DEV
    HALF = N // 2

    def body(a_ref, b_ref, out_ref, a_bf, b_bf,
             send_r, send_l, recv_r, recv_l,
             send_sems_r, send_sems_l, recv_sems_r, recv_sems_l):
        my = lax.axis_index("i")
        left = lax.rem(my + N_DEV - 1, N_DEV)
        right = lax.rem(my + 1, N_DEV)

        barrier_sem = pltpu.get_barrier_semaphore()
        for nbr in (left, right):
            pl.semaphore_signal(
                barrier_sem, inc=1,
                device_id=(nbr,), device_id_type=pl.DeviceIdType.MESH,
            )
        pl.semaphore_wait(barrier_sem, 2)

        a_bf[...] = a_ref[...].astype(jnp.bfloat16)
        b_bf[...] = b_ref[...].astype(jnp.bfloat16)

        def pchunk_r(c):
            a_c = a_bf[pl.ds(c * CHUNK, CHUNK), :]
            return lax.dot_general(
                a_c, b_bf[:, :HALF], (((1,), (0,)), ((), ())),
                preferred_element_type=jnp.float32,
            )

        def pchunk_l(c):
            a_c = a_bf[pl.ds(c * CHUNK, CHUNK), :]
            return lax.dot_general(
                a_c, b_bf[:, HALF:], (((1,), (0,)), ((), ())),
                preferred_element_type=jnp.float32,
            )

        def mk_rdma(direction, s):
            if direction == "r":
                return pltpu.make_async_remote_copy(
                    src_ref=send_r.at[s % 2], dst_ref=recv_r.at[s],
                    send_sem=send_sems_r.at[s % 2], recv_sem=recv_sems_r.at[s],
                    device_id=(right,), device_id_type=pl.DeviceIdType.MESH,
                )
            return pltpu.make_async_remote_copy(
                src_ref=send_l.at[s % 2], dst_ref=recv_l.at[s],
                send_sem=send_sems_l.at[s % 2], recv_sem=recv_sems_l.at[s],
                device_id=(left,), device_id_type=pl.DeviceIdType.MESH,
            )

        acc_r = pchunk_r(left)
        acc_l = pchunk_l(right)
        send_r[0] = acc_r.astype(jnp.bfloat16)
        send_l[0] = acc_l.astype(jnp.bfloat16)
        mk_rdma("r", 0).start()
        mk_rdma("l", 0).start()

        for s in range(N_DEV - 1):
            c_r = lax.rem(my + (2 * N_DEV - 2 - s), N_DEV)
            c_l = lax.rem(my + 2 + s, N_DEV)
            p_r = pchunk_r(c_r)
            p_l = pchunk_l(c_l)
            mk_rdma("r", s).wait_recv()
            mk_rdma("l", s).wait_recv()
            if s >= 1:
                mk_rdma("r", s - 1).wait_send()
                mk_rdma("l", s - 1).wait_send()
            if s < N_DEV - 2:
                acc_r = recv_r[s].astype(jnp.float32) + p_r
                acc_l = recv_l[s].astype(jnp.float32) + p_l
                send_r[(s + 1) % 2] = acc_r.astype(jnp.bfloat16)
                send_l[(s + 1) % 2] = acc_l.astype(jnp.bfloat16)
                mk_rdma("r", s + 1).start()
                mk_rdma("l", s + 1).start()
            else:
                out_ref[:, :HALF] = recv_r[s].astype(jnp.float32) + p_r
                out_ref[:, HALF:] = recv_l[s].astype(jnp.float32) + p_l

        mk_rdma("r", N_DEV - 2).wait_send()
        mk_rdma("l", N_DEV - 2).wait_send()

    return pl.pallas_call(
        body,
        out_shape=jax.ShapeDtypeStruct((CHUNK, N), jnp.float32),
        in_specs=[
            pl.BlockSpec(memory_space=pltpu.VMEM),
            pl.BlockSpec(memory_space=pltpu.VMEM),
        ],
        out_specs=pl.BlockSpec(memory_space=pltpu.VMEM),
        scratch_shapes=[
            pltpu.VMEM((M, K), jnp.bfloat16),
            pltpu.VMEM((K, N), jnp.bfloat16),
            pltpu.VMEM((2, CHUNK, HALF), jnp.bfloat16),
            pltpu.VMEM((2, CHUNK, HALF), jnp.bfloat16),
            pltpu.VMEM((N_DEV - 1, CHUNK, HALF), jnp.bfloat16),
            pltpu.VMEM((N_DEV - 1, CHUNK, HALF), jnp.bfloat16),
            pltpu.SemaphoreType.DMA((2,)),
            pltpu.SemaphoreType.DMA((2,)),
            pltpu.SemaphoreType.DMA((N_DEV - 1,)),
            pltpu.SemaphoreType.DMA((N_DEV - 1,)),
        ],
        compiler_params=pltpu.CompilerParams(collective_id=0),
    )(A, B)


# device time: 55082 ns/iter; 2.0853x vs baseline; 1.2906x over previous
import jax
import jax.numpy as jnp
from jax import lax
from jax.experimental import pallas as pl
from jax.experimental.pallas import tpu as pltpu

N_DEV = 8
SUB = 2


def kernel(A, B):
    M, K = A.shape
    K2, N = B.shape
    assert K == K2
    CHUNK = M // N_DEV
    HALF = N // 2
    ROWS = CHUNK // SUB

    def body(a_ref, b_ref, out_ref, b_bf,
             send_r, send_l, recv_r, recv_l,
             send_sems_r, send_sems_l, recv_sems_r, recv_sems_l):
        my = lax.axis_index("i")
        left = lax.rem(my + N_DEV - 1, N_DEV)
        right = lax.rem(my + 1, N_DEV)

        barrier_sem = pltpu.get_barrier_semaphore()
        for nbr in (left, right):
            pl.semaphore_signal(
                barrier_sem, inc=1,
                device_id=(nbr,), device_id_type=pl.DeviceIdType.MESH,
            )
        pl.semaphore_wait(barrier_sem, 2)

        def pchunk(c, lo, hi):
            a_c = a_ref[pl.ds(c * CHUNK, CHUNK), :].astype(jnp.bfloat16)
            return lax.dot_general(
                a_c, b_bf[:, lo:hi], (((1,), (0,)), ((), ())),
                preferred_element_type=jnp.float32,
            )

        def mk_rdma(direction, s, u):
            if direction == "r":
                return pltpu.make_async_remote_copy(
                    src_ref=send_r.at[s % 2, u], dst_ref=recv_r.at[s, u],
                    send_sem=send_sems_r.at[s % 2, u],
                    recv_sem=recv_sems_r.at[s, u],
                    device_id=(right,), device_id_type=pl.DeviceIdType.MESH,
                )
            return pltpu.make_async_remote_copy(
                src_ref=send_l.at[s % 2, u], dst_ref=recv_l.at[s, u],
                send_sem=send_sems_l.at[s % 2, u],
                recv_sem=recv_sems_l.at[s, u],
                device_id=(left,), device_id_type=pl.DeviceIdType.MESH,
            )

        def stage(direction, s, acc):
            buf = send_r if direction == "r" else send_l
            for u in range(SUB):
                buf[s % 2, u] = acc[u * ROWS:(u + 1) * ROWS, :].astype(
                    jnp.bfloat16)
                mk_rdma(direction, s, u).start()


        b_bf[:, :HALF] = b_ref[:, :HALF].astype(jnp.bfloat16)
        stage("r", 0, pchunk(left, 0, HALF))
        b_bf[:, HALF:] = b_ref[:, HALF:].astype(jnp.bfloat16)
        stage("l", 0, pchunk(right, HALF, N))

        for s in range(N_DEV - 1):
            c_r = lax.rem(my + (2 * N_DEV - 2 - s), N_DEV)
            c_l = lax.rem(my + 2 + s, N_DEV)
            p_r = pchunk(c_r, 0, HALF)
            p_l = pchunk(c_l, HALF, N)
            if s >= 1:
                for u in range(SUB):
                    mk_rdma("r", s - 1, u).wait_send()
                    mk_rdma("l", s - 1, u).wait_send()
            if s < N_DEV - 2:
                for u in range(SUB):
                    rows = slice(u * ROWS, (u + 1) * ROWS)
                    mk_rdma("r", s, u).wait_recv()
                    acc_u = recv_r[s, u].astype(jnp.float32) + p_r[rows, :]
                    send_r[(s + 1) % 2, u] = acc_u.astype(jnp.bfloat16)
                    mk_rdma("r", s + 1, u).start()
                    mk_rdma("l", s, u).wait_recv()
                    acc_u = recv_l[s, u].astype(jnp.float32) + p_l[rows, :]
                    send_l[(s + 1) % 2, u] = acc_u.astype(jnp.bfloat16)
                    mk_rdma("l", s + 1, u).start()
            else:
                for u in range(SUB):
                    rows = slice(u * ROWS, (u + 1) * ROWS)
                    mk_rdma("r", s, u).wait_recv()
                    out_ref[rows, :HALF] = (
                        recv_r[s, u].astype(jnp.float32) + p_r[rows, :])
                    mk_rdma("l", s, u).wait_recv()
                    out_ref[rows, HALF:] = (
                        recv_l[s, u].astype(jnp.float32) + p_l[rows, :])

        for u in range(SUB):
            mk_rdma("r", N_DEV - 2, u).wait_send()
            mk_rdma("l", N_DEV - 2, u).wait_send()

    return pl.pallas_call(
        body,
        out_shape=jax.ShapeDtypeStruct((CHUNK, N), jnp.float32),
        in_specs=[
            pl.BlockSpec(memory_space=pltpu.VMEM),
            pl.BlockSpec(memory_space=pltpu.VMEM),
        ],
        out_specs=pl.BlockSpec(memory_space=pltpu.VMEM),
        scratch_shapes=[
            pltpu.VMEM((K, N), jnp.bfloat16),
            pltpu.VMEM((2, SUB, ROWS, HALF), jnp.bfloat16),
            pltpu.VMEM((2, SUB, ROWS, HALF), jnp.bfloat16),
            pltpu.VMEM((N_DEV - 1, SUB, ROWS, HALF), jnp.bfloat16),
            pltpu.VMEM((N_DEV - 1, SUB, ROWS, HALF), jnp.bfloat16),
            pltpu.SemaphoreType.DMA((2, SUB)),
            pltpu.SemaphoreType.DMA((2, SUB)),
            pltpu.SemaphoreType.DMA((N_DEV - 1, SUB)),
            pltpu.SemaphoreType.DMA((N_DEV - 1, SUB)),
        ],
        compiler_params=pltpu.CompilerParams(collective_id=0),
    )(A, B)
